# baseline (speedup 1.0000x reference)
import functools

import jax
import jax.numpy as jnp
from jax.experimental import pallas as pl
from jax.experimental.pallas import tpu as pltpu


_RS = 512


def _mlp_kernel(x_ref, w1q_ref, w1_ref, w2_ref, b2_ref, o_ref):
    tb = x_ref.shape[0]
    rs = min(_RS, tb)
    n = tb // rs
    w1q = w1q_ref[...]
    w1 = w1_ref[...]
    w2 = w2_ref[...]
    b2 = b2_ref[...]

    kq = x_ref.shape[1] // 2

    def d1(i):
        xs = x_ref[i * rs:(i + 1) * rs, :]
        xq = xs[:, :kq].astype(jnp.float8_e4m3fn)
        xb = xs[:, kq:].astype(jnp.bfloat16)
        zq = jax.lax.dot_general(
            xq, w1q, (((1,), (1,)), ((), ())),
            preferred_element_type=jnp.float32)
        zb = jax.lax.dot_general(
            xb, w1, (((1,), (1,)), ((), ())),
            preferred_element_type=jnp.float32)
        return zq * (1.0 / 16.0) + zb

    def d2(h):
        t = jnp.tanh(h).astype(jnp.bfloat16)
        return jax.lax.dot_general(
            t, w2, (((1,), (1,)), ((), ())),
            preferred_element_type=jnp.float32)

    def fin(i, y):
        o_ref[i * rs:(i + 1) * rs, :] = 0.5 * jnp.tanh(y + b2) + 0.5

    h = [None] * n
    y = [None] * n
    h[0] = d1(0)
    if n > 1:
        h[1] = d1(1)
    y[0] = d2(h[0])
    for i in range(n):
        if i + 2 < n:
            h[i + 2] = d1(i + 2)
        if i + 1 < n:
            y[i + 1] = d2(h[i + 1])
        fin(i, y[i])
        h[i] = y[i] = None


@functools.partial(jax.jit, static_argnames=("batch_block",))
def _mlp_forward(x, w1, w2, batch_block=1024):
    batch, input_size = x.shape
    hidden_size, _ = w1.shape
    output_size, _ = w2.shape

    kq = input_size // 2
    w1q = (8.0 * w1[:, :kq]).astype(jnp.float8_e4m3fn)
    w1b = (0.5 * w1[:, kq:]).astype(jnp.bfloat16)
    w2b = (0.25 * w2).astype(jnp.bfloat16)
    b2 = jnp.sum(w2b, axis=1, dtype=jnp.float32).reshape(1, output_size)

    n_blocks = pl.cdiv(batch, batch_block)
    padded_batch = n_blocks * batch_block
    if padded_batch != batch:
        x = jnp.pad(x, ((0, padded_batch - batch), (0, 0)))

    out = pl.pallas_call(
        _mlp_kernel,
        out_shape=jax.ShapeDtypeStruct((padded_batch, output_size), jnp.float32),
        grid=(n_blocks,),
        in_specs=[
            pl.BlockSpec((batch_block, input_size), lambda i: (i, 0)),
            pl.BlockSpec((hidden_size, kq), lambda i: (0, 0)),
            pl.BlockSpec((hidden_size, input_size - kq), lambda i: (0, 0)),
            pl.BlockSpec((output_size, hidden_size), lambda i: (0, 0)),
            pl.BlockSpec((1, output_size), lambda i: (0, 0)),
        ],
        out_specs=pl.BlockSpec((batch_block, output_size), lambda i: (i, 0)),
        compiler_params=pltpu.CompilerParams(
            dimension_semantics=("parallel",),
        ),
    )(x, w1q, w1b, w2b, b2)

    if padded_batch != batch:
        out = out[:batch]
    return out


def kernel(x, w1, w2):
    return _mlp_forward(x, w1, w2)
